# baseline (speedup 1.0000x reference)
import jax
import jax.numpy as jnp
from jax import lax
from jax.experimental import pallas as pl
from jax.experimental.pallas import tpu as pltpu
from jax.experimental.pallas import tpu_sc as plsc

_B, _N, _E = 2, 100000, 3200000
_NPAD = 102400
_WN = 10240
_WE = 128000

_NC, _NS = 2, 16
_NW = _NC * _NS
_K = 6400
_M = _E // _K
_SLICE = _NPAD // _NS
_NSEG = _NC * _B

_INTERPRET = False


def _prep_kernel(v, th, a0, a1, b0, b1):
    vv = v[...]
    t = th[...]
    ca = vv * jnp.cos(t)
    sa = vv * jnp.sin(t)
    a0[...] = ca[0]
    a1[...] = ca[1]
    b0[...] = sa[0]
    b1[...] = sa[1]


def _run_prep(vpad, apad):
    out = jax.ShapeDtypeStruct((_NPAD,), jnp.float32)
    return pl.pallas_call(
        _prep_kernel,
        out_shape=[out, out, out, out],
        interpret=_INTERPRET,
    )(vpad, apad)


def _sc_flow_body(a0, a1, b0, b1, src, dst, g, su,
                  pout, qout,
                  ap0, ap1, aq0, aq1,
                  srcv, dstv, gv2, aiv, ajv, biv, bjv,
                  pv, npv, qv, nqv, zv):
    c = lax.axis_index("c")
    s = lax.axis_index("s")
    wid = c * _NS + s
    sl = pl.ds(s * _SLICE, _SLICE)

    def zero16(r, _):
        zv[pl.ds(r * 16, 16)] = jnp.zeros((16,), jnp.float32)
        return _
    lax.fori_loop(0, _SLICE // 16, zero16, None)
    for acc in (ap0, ap1, aq0, aq1):
        pltpu.sync_copy(zv, acc.at[sl])
    plsc.subcore_barrier()

    def chunk(t, carry):
        e0 = (wid + t * _NW) * _K
        ds_e = pl.ds(e0, _K)
        pltpu.sync_copy(src.at[ds_e], srcv)
        pltpu.sync_copy(dst.at[ds_e], dstv)
        pltpu.sync_copy(g.at[:, ds_e], gv2.at[0])
        pltpu.sync_copy(su.at[:, ds_e], gv2.at[1])
        for b, (ta, tb, ap, aq) in enumerate(
                ((a0, b0, ap0, aq0), (a1, b1, ap1, aq1))):
            pltpu.sync_copy(ta.at[srcv], aiv)
            pltpu.sync_copy(ta.at[dstv], ajv)
            pltpu.sync_copy(tb.at[srcv], biv)
            pltpu.sync_copy(tb.at[dstv], bjv)

            def vec(r, carry2):
                d16 = pl.ds(r * 16, 16)
                ai_, aj_ = aiv[d16], ajv[d16]
                bi2, bj2 = biv[d16], bjv[d16]
                cc = ai_ * aj_ + bi2 * bj2
                ss = bi2 * aj_ - ai_ * bj2
                g_, b_ = gv2[0, b, d16], gv2[1, b, d16]
                p_ = g_ * cc + b_ * ss
                q_ = g_ * ss - b_ * cc
                pv[d16] = p_
                npv[d16] = -p_
                qv[d16] = q_
                nqv[d16] = -q_
                return carry2
            lax.fori_loop(0, _K // 16, vec, None)
            pltpu.sync_copy(pv, ap.at[srcv], add=True)
            pltpu.sync_copy(npv, ap.at[dstv], add=True)
            pltpu.sync_copy(qv, aq.at[srcv], add=True)
            pltpu.sync_copy(nqv, aq.at[dstv], add=True)
        return carry
    nchunks = (_M - wid + _NW - 1) // _NW
    lax.fori_loop(0, nchunks, chunk, None)

    plsc.subcore_barrier()
    cbase = c * _B * _NPAD + s * _SLICE
    for k, acc in enumerate((ap0, ap1)):
        pltpu.sync_copy(acc.at[sl], pout.at[pl.ds(cbase + k * _NPAD, _SLICE)])
    for k, acc in enumerate((aq0, aq1)):
        pltpu.sync_copy(acc.at[sl], qout.at[pl.ds(cbase + k * _NPAD, _SLICE)])


def _run_sc_flow(a0, a1, b0, b1, edge_index, conductance, susceptance):
    mesh = plsc.VectorSubcoreMesh(core_axis_name="c", subcore_axis_name="s",
                                  num_cores=_NC, num_subcores=_NS)
    f = pl.kernel(
        _sc_flow_body,
        out_type=[jax.ShapeDtypeStruct((_NSEG * _NPAD,), jnp.float32),
                  jax.ShapeDtypeStruct((_NSEG * _NPAD,), jnp.float32)],
        mesh=mesh,
        scratch_types=[pltpu.VMEM_SHARED((_NPAD,), jnp.float32)] * 4
                      + [pltpu.VMEM((_K,), jnp.int32)] * 2
                      + [pltpu.VMEM((2, _B, _K), jnp.float32)]
                      + [pltpu.VMEM((_K,), jnp.float32)] * 8
                      + [pltpu.VMEM((_SLICE,), jnp.float32)],
        interpret=_INTERPRET,
    )
    return f(a0, a1, b0, b1, edge_index[0], edge_index[1],
             conductance, susceptance)


def _node_loss_kernel(fp, fl, v, tgt, pinj, qinj, freq, pimb, *refs):
    pseg = refs[0:4]
    qseg = refs[4:8]
    out = refs[8]
    i = pl.program_id(0)
    col = lax.broadcasted_iota(jnp.int32, (1, _WN), 1) + i * _WN
    m = col < _N

    @pl.when(i == 0)
    def _init():
        for k in range(5):
            out[k] = 0.0
        ef = 60.0 + 6.0 * pimb[...]
        out[5] = jnp.sum((freq[...] - ef) ** 2)

    p = jnp.clip(fp[...], 1e-6, 1.0 - 1e-6)
    bce = -(fl[...] * jnp.log(p) + (1.0 - fl[...]) * jnp.log(1.0 - p))
    out[0] += jnp.sum(jnp.where(m, bce, 0.0))

    def calc(seg):
        b0 = seg[0][...] + seg[2][...]
        b1 = seg[1][...] + seg[3][...]
        return jnp.stack([b0, b1])

    out[1] += jnp.sum(jnp.where(m, (calc(pseg) - pinj[...]) ** 2, 0.0))
    out[2] += jnp.sum(jnp.where(m, (calc(qseg) - qinj[...]) ** 2, 0.0))

    vv = v[...]
    low = jnp.maximum(0.95 - vv, 0.0)
    high = jnp.maximum(vv - 1.05, 0.0)
    out[3] += jnp.sum(jnp.where(m, low * low + high * high, 0.0))
    out[4] += jnp.sum(jnp.where(m, (vv - tgt[...]) ** 2, 0.0))


def _run_node_loss(fp, fl, v, tgt, pinj, qinj, freq, pimb, pflat, qflat):
    spec_n = pl.BlockSpec((_B, _WN), lambda i: (0, i))
    spec_s = pl.BlockSpec((_B, 1), lambda i: (0, 0))
    nseg_blocks = _NPAD // _WN
    seg_specs = [pl.BlockSpec((_WN,), lambda i, k=k: (k * nseg_blocks + i,))
                 for k in range(_NSEG)]
    return pl.pallas_call(
        _node_loss_kernel,
        grid=(_NPAD // _WN,),
        in_specs=[spec_n] * 6 + [spec_s, spec_s] + seg_specs + seg_specs,
        out_specs=pl.BlockSpec(memory_space=pltpu.SMEM),
        out_shape=jax.ShapeDtypeStruct((6,), jnp.float32),
        interpret=_INTERPRET,
    )(fp, fl, v, tgt, pinj, qinj, freq, pimb,
      *([pflat] * _NSEG), *([qflat] * _NSEG))


def _edge_loss_kernel(lf, tl, out):
    i = pl.program_id(0)

    @pl.when(i == 0)
    def _init():
        out[0] = 0.0

    viol = jnp.maximum(jnp.abs(lf[...]) - tl[...], 0.0)
    out[0] += jnp.sum(viol * viol)


def _run_edge_loss(lf, tl):
    spec_e = pl.BlockSpec((_B, _WE), lambda i: (0, i))
    return pl.pallas_call(
        _edge_loss_kernel,
        grid=(_E // _WE,),
        in_specs=[spec_e, spec_e],
        out_specs=pl.BlockSpec(memory_space=pltpu.SMEM),
        out_shape=jax.ShapeDtypeStruct((1,), jnp.float32),
        interpret=_INTERPRET,
    )(lf, tl)


def kernel(failure_probability, failure_label, voltages, angles, line_flows,
           frequency, target_voltages, conductance, susceptance,
           power_injection, thermal_limits, reactive_injection,
           power_imbalance, edge_index):
    v = voltages[..., 0]
    th = angles[..., 0]
    fp = failure_probability[..., 0]
    fl = failure_label[..., 0]
    tgt = target_voltages[..., 0]
    pinj = power_injection[..., 0]
    qinj = reactive_injection[..., 0]
    lf = line_flows[..., 0]

    pad = ((0, 0), (0, _NPAD - _N))
    a0, a1, b0, b1 = _run_prep(jnp.pad(v, pad), jnp.pad(th, pad))

    pflat, qflat = _run_sc_flow(a0, a1, b0, b1,
                                edge_index, conductance, susceptance)

    sums = _run_node_loss(fp, fl, v, tgt, pinj, qinj,
                          frequency, power_imbalance, pflat, qflat)
    cap = _run_edge_loss(lf, thermal_limits)

    bn = float(_B * _N)
    total = (sums[0] / bn
             + 0.1 * (sums[1] / bn)
             + 0.05 * (cap[0] / float(_B * _E))
             + 0.05 * (sums[3] / bn)
             + 0.08 * (sums[5] / float(_B))
             + 1.0 * (sums[4] / bn)
             + 0.1 * (sums[2] / bn))
    return total


# speedup vs baseline: 1.6568x; 1.6568x over previous
import jax
import jax.numpy as jnp
from jax import lax
from jax.experimental import pallas as pl
from jax.experimental.pallas import tpu as pltpu
from jax.experimental.pallas import tpu_sc as plsc

_B, _N, _E = 2, 100000, 3200000
_NPAD = 102400
_WN = 10240
_WE = 128000

_NC, _NS = 2, 16
_NW = _NC * _NS
_K = 5120
_M = _E // _K
_SLICE = _NPAD // _NS
_NSEG = _NC * _B

_INTERPRET = False


def _prep_kernel(v, th, a0, a1, b0, b1):
    vv = v[...]
    t = th[...]
    ca = vv * jnp.cos(t)
    sa = vv * jnp.sin(t)
    a0[...] = ca[0]
    a1[...] = ca[1]
    b0[...] = sa[0]
    b1[...] = sa[1]


def _run_prep(vpad, apad):
    out = jax.ShapeDtypeStruct((_NPAD,), jnp.float32)
    return pl.pallas_call(
        _prep_kernel,
        out_shape=[out, out, out, out],
        interpret=_INTERPRET,
    )(vpad, apad)


def _sc_flow_body(a0, a1, b0, b1, src, dst, g, su,
                  pout, qout,
                  ta0, ta1, tb0, tb1, ap0, ap1, aq0, aq1,
                  srcv, dstv, gv2, aiv, ajv, biv, bjv,
                  pv, npv, qv, nqv, zv):
    c = lax.axis_index("c")
    s = lax.axis_index("s")
    wid = c * _NS + s
    sl = pl.ds(s * _SLICE, _SLICE)

    def zero16(r, _):
        zv[pl.ds(r * 16, 16)] = jnp.zeros((16,), jnp.float32)
        return _
    lax.fori_loop(0, _SLICE // 16, zero16, None)
    for acc in (ap0, ap1, aq0, aq1):
        pltpu.sync_copy(zv, acc.at[sl])
    for hbm, tab in ((a0, ta0), (a1, ta1), (b0, tb0), (b1, tb1)):
        pltpu.sync_copy(hbm.at[sl], tab.at[sl])
    plsc.subcore_barrier()

    def chunk(t, carry):
        e0 = (wid + t * _NW) * _K
        ds_e = pl.ds(e0, _K)
        pltpu.sync_copy(src.at[ds_e], srcv)
        pltpu.sync_copy(dst.at[ds_e], dstv)
        pltpu.sync_copy(g.at[:, ds_e], gv2.at[0])
        pltpu.sync_copy(su.at[:, ds_e], gv2.at[1])
        for b, (ta, tb, ap, aq) in enumerate(
                ((ta0, tb0, ap0, aq0), (ta1, tb1, ap1, aq1))):
            pltpu.sync_copy(ta.at[srcv], aiv)
            pltpu.sync_copy(ta.at[dstv], ajv)
            pltpu.sync_copy(tb.at[srcv], biv)
            pltpu.sync_copy(tb.at[dstv], bjv)

            def vec(r, carry2):
                d16 = pl.ds(r * 16, 16)
                ai_, aj_ = aiv[d16], ajv[d16]
                bi2, bj2 = biv[d16], bjv[d16]
                cc = ai_ * aj_ + bi2 * bj2
                ss = bi2 * aj_ - ai_ * bj2
                g_, b_ = gv2[0, b, d16], gv2[1, b, d16]
                p_ = g_ * cc + b_ * ss
                q_ = g_ * ss - b_ * cc
                pv[d16] = p_
                npv[d16] = -p_
                qv[d16] = q_
                nqv[d16] = -q_
                return carry2
            lax.fori_loop(0, _K // 16, vec, None)
            pltpu.sync_copy(pv, ap.at[srcv], add=True)
            pltpu.sync_copy(npv, ap.at[dstv], add=True)
            pltpu.sync_copy(qv, aq.at[srcv], add=True)
            pltpu.sync_copy(nqv, aq.at[dstv], add=True)
        return carry
    nchunks = (_M - wid + _NW - 1) // _NW
    lax.fori_loop(0, nchunks, chunk, None)

    plsc.subcore_barrier()
    pltpu.sync_copy(ap0.at[sl], pout.at[c, 0, sl])
    pltpu.sync_copy(ap1.at[sl], pout.at[c, 1, sl])
    pltpu.sync_copy(aq0.at[sl], qout.at[c, 0, sl])
    pltpu.sync_copy(aq1.at[sl], qout.at[c, 1, sl])


def _run_sc_flow(a0, a1, b0, b1, edge_index, conductance, susceptance):
    mesh = plsc.VectorSubcoreMesh(core_axis_name="c", subcore_axis_name="s",
                                  num_cores=_NC, num_subcores=_NS)
    f = pl.kernel(
        _sc_flow_body,
        out_type=[jax.ShapeDtypeStruct((_NC, _B, _NPAD), jnp.float32),
                  jax.ShapeDtypeStruct((_NC, _B, _NPAD), jnp.float32)],
        mesh=mesh,
        scratch_types=[pltpu.VMEM_SHARED((_NPAD,), jnp.float32)] * 8
                      + [pltpu.VMEM((_K,), jnp.int32)] * 2
                      + [pltpu.VMEM((2, _B, _K), jnp.float32)]
                      + [pltpu.VMEM((_K,), jnp.float32)] * 8
                      + [pltpu.VMEM((_SLICE,), jnp.float32)],
        interpret=_INTERPRET,
    )
    return f(a0, a1, b0, b1, edge_index[0], edge_index[1],
             conductance, susceptance)


def _node_loss_kernel(fp, fl, v, tgt, pinj, qinj, freq, pimb, pp, qp, out):
    i = pl.program_id(0)
    col = lax.broadcasted_iota(jnp.int32, (1, _WN), 1) + i * _WN
    m = col < _N

    @pl.when(i == 0)
    def _init():
        for k in range(5):
            out[k] = 0.0
        ef = 60.0 + 6.0 * pimb[...]
        out[5] = jnp.sum((freq[...] - ef) ** 2)

    p = jnp.clip(fp[...], 1e-6, 1.0 - 1e-6)
    bce = -(fl[...] * jnp.log(p) + (1.0 - fl[...]) * jnp.log(1.0 - p))
    out[0] += jnp.sum(jnp.where(m, bce, 0.0))

    p_calc = jnp.sum(pp[...], axis=0)
    out[1] += jnp.sum(jnp.where(m, (p_calc - pinj[...]) ** 2, 0.0))
    q_calc = jnp.sum(qp[...], axis=0)
    out[2] += jnp.sum(jnp.where(m, (q_calc - qinj[...]) ** 2, 0.0))

    vv = v[...]
    low = jnp.maximum(0.95 - vv, 0.0)
    high = jnp.maximum(vv - 1.05, 0.0)
    out[3] += jnp.sum(jnp.where(m, low * low + high * high, 0.0))
    out[4] += jnp.sum(jnp.where(m, (vv - tgt[...]) ** 2, 0.0))


def _run_node_loss(fp, fl, v, tgt, pinj, qinj, freq, pimb, pp, qp):
    spec_n = pl.BlockSpec((_B, _WN), lambda i: (0, i))
    spec_s = pl.BlockSpec((_B, 1), lambda i: (0, 0))
    spec_p = pl.BlockSpec((_NC, _B, _WN), lambda i: (0, 0, i))
    return pl.pallas_call(
        _node_loss_kernel,
        grid=(_NPAD // _WN,),
        in_specs=[spec_n] * 6 + [spec_s, spec_s, spec_p, spec_p],
        out_specs=pl.BlockSpec(memory_space=pltpu.SMEM),
        out_shape=jax.ShapeDtypeStruct((6,), jnp.float32),
        interpret=_INTERPRET,
    )(fp, fl, v, tgt, pinj, qinj, freq, pimb, pp, qp)


def _edge_loss_kernel(lf, tl, out):
    i = pl.program_id(0)

    @pl.when(i == 0)
    def _init():
        out[0] = 0.0

    viol = jnp.maximum(jnp.abs(lf[...]) - tl[...], 0.0)
    out[0] += jnp.sum(viol * viol)


def _run_edge_loss(lf, tl):
    spec_e = pl.BlockSpec((_B, _WE), lambda i: (0, i))
    return pl.pallas_call(
        _edge_loss_kernel,
        grid=(_E // _WE,),
        in_specs=[spec_e, spec_e],
        out_specs=pl.BlockSpec(memory_space=pltpu.SMEM),
        out_shape=jax.ShapeDtypeStruct((1,), jnp.float32),
        interpret=_INTERPRET,
    )(lf, tl)


def kernel(failure_probability, failure_label, voltages, angles, line_flows,
           frequency, target_voltages, conductance, susceptance,
           power_injection, thermal_limits, reactive_injection,
           power_imbalance, edge_index):
    v = voltages[..., 0]
    th = angles[..., 0]
    fp = failure_probability[..., 0]
    fl = failure_label[..., 0]
    tgt = target_voltages[..., 0]
    pinj = power_injection[..., 0]
    qinj = reactive_injection[..., 0]
    lf = line_flows[..., 0]

    pad = ((0, 0), (0, _NPAD - _N))
    a0, a1, b0, b1 = _run_prep(jnp.pad(v, pad), jnp.pad(th, pad))

    pflat, qflat = _run_sc_flow(a0, a1, b0, b1,
                                edge_index, conductance, susceptance)

    sums = _run_node_loss(fp, fl, v, tgt, pinj, qinj,
                          frequency, power_imbalance, pflat, qflat)
    cap = _run_edge_loss(lf, thermal_limits)

    bn = float(_B * _N)
    total = (sums[0] / bn
             + 0.1 * (sums[1] / bn)
             + 0.05 * (cap[0] / float(_B * _E))
             + 0.05 * (sums[3] / bn)
             + 0.08 * (sums[5] / float(_B))
             + 1.0 * (sums[4] / bn)
             + 0.1 * (sums[2] / bn))
    return total


# speedup vs baseline: 2.0759x; 1.2530x over previous
import jax
import jax.numpy as jnp
from jax import lax
from jax.experimental import pallas as pl
from jax.experimental.pallas import tpu as pltpu
from jax.experimental.pallas import tpu_sc as plsc

_B, _N, _E = 2, 100000, 3200000
_NPAD = 102400
_WN = 10240
_WE = 128000

_NC, _NS = 2, 16
_NW = _NC * _NS
_K = 5120
_M = _E // _K
_SLICE = _NPAD // _NS
_NSEG = _NC * _B

_INTERPRET = False


def _prep_kernel(v, th, ei, a0, a1, b0, b1, srco, dsto):
    i = pl.program_id(0)

    @pl.when(i == 0)
    def _trig():
        vv = v[...]
        tt = th[...]
        ca = vv * jnp.cos(tt)
        sa = vv * jnp.sin(tt)
        a0[...] = ca[0]
        a1[...] = ca[1]
        b0[...] = sa[0]
        b1[...] = sa[1]

    srco[...] = ei[0]
    dsto[...] = ei[1]


def _run_prep(vpad, apad, edge_index):
    outf = jax.ShapeDtypeStruct((_NPAD,), jnp.float32)
    outi = jax.ShapeDtypeStruct((_E,), jnp.int32)
    spec_v = pl.BlockSpec((_B, _NPAD), lambda i: (0, 0))
    spec_f = pl.BlockSpec((_NPAD,), lambda i: (0,))
    spec_ei = pl.BlockSpec((2, _WE), lambda i: (0, i))
    spec_o = pl.BlockSpec((_WE,), lambda i: (i,))
    return pl.pallas_call(
        _prep_kernel,
        grid=(_E // _WE,),
        in_specs=[spec_v, spec_v, spec_ei],
        out_specs=[spec_f] * 4 + [spec_o, spec_o],
        out_shape=[outf, outf, outf, outf, outi, outi],
        interpret=_INTERPRET,
    )(vpad, apad, edge_index)


def _sc_flow_body(a0, a1, b0, b1, src, dst, g, su,
                  pout, qout,
                  ta0, ta1, tb0, tb1, ap0, ap1, aq0, aq1,
                  srcv, dstv, gv2, aiv, ajv, biv, bjv,
                  pv, npv, qv, nqv, zv, semg, semsc):
    c = lax.axis_index("c")
    s = lax.axis_index("s")
    wid = c * _NS + s
    sl = pl.ds(s * _SLICE, _SLICE)

    def zero16(r, _):
        zv[pl.ds(r * 16, 16)] = jnp.zeros((16,), jnp.float32)
        return _
    lax.fori_loop(0, _SLICE // 16, zero16, None)
    for acc in (ap0, ap1, aq0, aq1):
        pltpu.sync_copy(zv, acc.at[sl])
    for hbm, tab in ((a0, ta0), (a1, ta1), (b0, tb0), (b1, tb1)):
        pltpu.sync_copy(hbm.at[sl], tab.at[sl])
    plsc.subcore_barrier()

    def chunk(t, carry):
        e0 = (wid + t * _NW) * _K
        ds_e = pl.ds(e0, _K)
        pltpu.sync_copy(src.at[ds_e], srcv)
        pltpu.sync_copy(dst.at[ds_e], dstv)
        pltpu.sync_copy(g.at[:, ds_e], gv2.at[0])
        pltpu.sync_copy(su.at[:, ds_e], gv2.at[1])
        scat = []
        for b, (ta, tb, ap, aq) in enumerate(
                ((ta0, tb0, ap0, aq0), (ta1, tb1, ap1, aq1))):
            gath = [pltpu.async_copy(ta.at[srcv], aiv, semg),
                    pltpu.async_copy(ta.at[dstv], ajv, semg),
                    pltpu.async_copy(tb.at[srcv], biv, semg),
                    pltpu.async_copy(tb.at[dstv], bjv, semg)]
            for cp in gath:
                cp.wait()
            for cp in scat:
                cp.wait()
            scat = []

            def vec(r, carry2):
                d16 = pl.ds(r * 16, 16)
                ai_, aj_ = aiv[d16], ajv[d16]
                bi2, bj2 = biv[d16], bjv[d16]
                cc = ai_ * aj_ + bi2 * bj2
                ss = bi2 * aj_ - ai_ * bj2
                g_, b_ = gv2[0, b, d16], gv2[1, b, d16]
                p_ = g_ * cc + b_ * ss
                q_ = g_ * ss - b_ * cc
                pv[d16] = p_
                npv[d16] = -p_
                qv[d16] = q_
                nqv[d16] = -q_
                return carry2
            lax.fori_loop(0, _K // 16, vec, None)
            scat = [pltpu.async_copy(pv, ap.at[srcv], semsc, add=True),
                    pltpu.async_copy(npv, ap.at[dstv], semsc, add=True),
                    pltpu.async_copy(qv, aq.at[srcv], semsc, add=True),
                    pltpu.async_copy(nqv, aq.at[dstv], semsc, add=True)]
        for cp in scat:
            cp.wait()
        return carry
    nchunks = (_M - wid + _NW - 1) // _NW
    lax.fori_loop(0, nchunks, chunk, None)

    plsc.subcore_barrier()
    pltpu.sync_copy(ap0.at[sl], pout.at[c, 0, sl])
    pltpu.sync_copy(ap1.at[sl], pout.at[c, 1, sl])
    pltpu.sync_copy(aq0.at[sl], qout.at[c, 0, sl])
    pltpu.sync_copy(aq1.at[sl], qout.at[c, 1, sl])


def _run_sc_flow(a0, a1, b0, b1, srcx, dstx, conductance, susceptance):
    mesh = plsc.VectorSubcoreMesh(core_axis_name="c", subcore_axis_name="s",
                                  num_cores=_NC, num_subcores=_NS)
    f = pl.kernel(
        _sc_flow_body,
        out_type=[jax.ShapeDtypeStruct((_NC, _B, _NPAD), jnp.float32),
                  jax.ShapeDtypeStruct((_NC, _B, _NPAD), jnp.float32)],
        mesh=mesh,
        scratch_types=[pltpu.VMEM_SHARED((_NPAD,), jnp.float32)] * 8
                      + [pltpu.VMEM((_K,), jnp.int32)] * 2
                      + [pltpu.VMEM((2, _B, _K), jnp.float32)]
                      + [pltpu.VMEM((_K,), jnp.float32)] * 8
                      + [pltpu.VMEM((_SLICE,), jnp.float32)]
                      + [pltpu.SemaphoreType.DMA] * 2,
        interpret=_INTERPRET,
    )
    return f(a0, a1, b0, b1, srcx, dstx, conductance, susceptance)


def _node_loss_kernel(fp, fl, v, tgt, pinj, qinj, freq, pimb, pp, qp, out):
    i = pl.program_id(0)
    col = lax.broadcasted_iota(jnp.int32, (1, _WN), 1) + i * _WN
    m = col < _N

    @pl.when(i == 0)
    def _init():
        for k in range(5):
            out[k] = 0.0
        ef = 60.0 + 6.0 * pimb[...]
        out[5] = jnp.sum((freq[...] - ef) ** 2)

    p = jnp.clip(fp[...], 1e-6, 1.0 - 1e-6)
    bce = -(fl[...] * jnp.log(p) + (1.0 - fl[...]) * jnp.log(1.0 - p))
    out[0] += jnp.sum(jnp.where(m, bce, 0.0))

    p_calc = jnp.sum(pp[...], axis=0)
    out[1] += jnp.sum(jnp.where(m, (p_calc - pinj[...]) ** 2, 0.0))
    q_calc = jnp.sum(qp[...], axis=0)
    out[2] += jnp.sum(jnp.where(m, (q_calc - qinj[...]) ** 2, 0.0))

    vv = v[...]
    low = jnp.maximum(0.95 - vv, 0.0)
    high = jnp.maximum(vv - 1.05, 0.0)
    out[3] += jnp.sum(jnp.where(m, low * low + high * high, 0.0))
    out[4] += jnp.sum(jnp.where(m, (vv - tgt[...]) ** 2, 0.0))


def _run_node_loss(fp, fl, v, tgt, pinj, qinj, freq, pimb, pp, qp):
    spec_n = pl.BlockSpec((_B, _WN), lambda i: (0, i))
    spec_s = pl.BlockSpec((_B, 1), lambda i: (0, 0))
    spec_p = pl.BlockSpec((_NC, _B, _WN), lambda i: (0, 0, i))
    return pl.pallas_call(
        _node_loss_kernel,
        grid=(_NPAD // _WN,),
        in_specs=[spec_n] * 6 + [spec_s, spec_s, spec_p, spec_p],
        out_specs=pl.BlockSpec(memory_space=pltpu.SMEM),
        out_shape=jax.ShapeDtypeStruct((6,), jnp.float32),
        interpret=_INTERPRET,
    )(fp, fl, v, tgt, pinj, qinj, freq, pimb, pp, qp)


def _edge_loss_kernel(lf, tl, out):
    i = pl.program_id(0)

    @pl.when(i == 0)
    def _init():
        out[0] = 0.0

    viol = jnp.maximum(jnp.abs(lf[...]) - tl[...], 0.0)
    out[0] += jnp.sum(viol * viol)


def _run_edge_loss(lf, tl):
    spec_e = pl.BlockSpec((_B, _WE), lambda i: (0, i))
    return pl.pallas_call(
        _edge_loss_kernel,
        grid=(_E // _WE,),
        in_specs=[spec_e, spec_e],
        out_specs=pl.BlockSpec(memory_space=pltpu.SMEM),
        out_shape=jax.ShapeDtypeStruct((1,), jnp.float32),
        interpret=_INTERPRET,
    )(lf, tl)


def kernel(failure_probability, failure_label, voltages, angles, line_flows,
           frequency, target_voltages, conductance, susceptance,
           power_injection, thermal_limits, reactive_injection,
           power_imbalance, edge_index):
    v = voltages[..., 0]
    th = angles[..., 0]
    fp = failure_probability[..., 0]
    fl = failure_label[..., 0]
    tgt = target_voltages[..., 0]
    pinj = power_injection[..., 0]
    qinj = reactive_injection[..., 0]
    lf = line_flows[..., 0]

    pad = ((0, 0), (0, _NPAD - _N))
    a0, a1, b0, b1, srcx, dstx = _run_prep(jnp.pad(v, pad), jnp.pad(th, pad),
                                           edge_index)

    pp, qp = _run_sc_flow(a0, a1, b0, b1, srcx, dstx,
                          conductance, susceptance)

    sums = _run_node_loss(fp, fl, v, tgt, pinj, qinj,
                          frequency, power_imbalance, pp, qp)
    cap = _run_edge_loss(lf, thermal_limits)

    bn = float(_B * _N)
    total = (sums[0] / bn
             + 0.1 * (sums[1] / bn)
             + 0.05 * (cap[0] / float(_B * _E))
             + 0.05 * (sums[3] / bn)
             + 0.08 * (sums[5] / float(_B))
             + 1.0 * (sums[4] / bn)
             + 0.1 * (sums[2] / bn))
    return total
